# baseline (speedup 1.0000x reference)
import jax
import jax.numpy as jnp
from jax.experimental import pallas as pl
from jax.experimental.pallas import tpu as pltpu

_K = 8


def _basis_rows(x, y, z):
    xx = x * x; yy = y * y; zz = z * z
    xy = x * y; yz = y * z; xz = x * z
    return [
        -0.4886025119029199 * y,
        0.4886025119029199 * z,
        -0.4886025119029199 * x,
        1.0925484305920792 * xy,
        -1.0925484305920792 * yz,
        0.31539156525252005 * (2.0 * zz - xx - yy),
        -1.0925484305920792 * xz,
        0.5462742152960396 * (xx - yy),
        -0.5900435899266435 * y * (3.0 * xx - yy),
        2.890611442640554 * xy * z,
        -0.4570457994644658 * y * (4.0 * zz - xx - yy),
        0.3731763325901154 * z * (2.0 * zz - 3.0 * xx - 3.0 * yy),
        -0.4570457994644658 * x * (4.0 * zz - xx - yy),
        1.445305721320277 * z * (xx - yy),
        -0.5900435899266435 * x * (xx - yy - zz),
    ]


def _body(order_ref, cnt_ref, camT_ref, *refs):
    k = (len(refs) - 5) // 2
    ins = refs[:2 * k]
    cx_ref, csc_ref, crt_ref, col_ref, cop_ref = refs[2 * k:]
    i = pl.program_id(0)
    camT = camT_ref[...]
    for j in range(k):
        pk_ref, shr_ref = ins[2 * j:2 * j + 2]
        valid = (k * i + j) < cnt_ref[0]
        pk = pk_ref[0]
        xyzT = pk[0:3]
        rotT = pk[6:10]

        cscT = jnp.exp(pk[3:6])
        qn = jnp.sqrt(jnp.sum(rotT * rotT, axis=0, keepdims=True))
        crtT = rotT * (1.0 / (qn + 1e-8))
        copT = 1.0 / (1.0 + jnp.exp(-pk[10:11]))

        d = xyzT - camT
        dn = jnp.sqrt(jnp.sum(d * d, axis=0, keepdims=True))
        dirs = d * (1.0 / (dn + 1e-8))
        x = dirs[0:1]; y = dirs[1:2]; z = dirs[2:3]
        basis = jnp.concatenate(_basis_rows(x, y, z), axis=0)
        col_rows = [
            jnp.sum(basis * shr_ref[ch], axis=0, keepdims=True)
            + 0.28209479177387814 * pk[11 + ch:12 + ch]
            for ch in range(3)
        ]
        colT = jnp.maximum(jnp.concatenate(col_rows, axis=0) + 0.5, 0.0)

        cx_ref[j] = jnp.where(valid, xyzT, 0.0)
        csc_ref[j] = jnp.where(valid, cscT, 1.0)
        crt_ref[j] = jnp.where(valid, crtT, 0.0)
        col_ref[j] = jnp.where(valid, colT, 0.5)
        cop_ref[j] = jnp.where(valid, copT, 0.5)


def _gather_compute(order, cnt, camT, packed, shr_v):
    c, _, cs = packed.shape
    k = _K if c % _K == 0 else 1

    chunk_specs = []
    ins = []
    for j in range(k):
        idx3 = (lambda jj: lambda i, o, n: (o[k * i + jj], 0, 0))(j)
        idx_shr = (lambda jj: lambda i, o, n: (0, 0, o[k * i + jj]))(j)
        chunk_specs += [
            pl.BlockSpec((1, 14, cs), idx3),
            pl.BlockSpec((3, 15, cs), idx_shr),
        ]
        ins += [packed, shr_v]

    in_specs = [pl.BlockSpec((3, cs), lambda i, o, n: (0, 0))] + chunk_specs
    out_specs = [pl.BlockSpec((k, d, cs), lambda i, o, n: (i, 0, 0))
                 for d in (3, 3, 4, 3, 1)]
    out_shapes = [jax.ShapeDtypeStruct((c, d, cs), jnp.float32)
                  for d in (3, 3, 4, 3, 1)]

    grid_spec = pltpu.PrefetchScalarGridSpec(
        num_scalar_prefetch=2,
        grid=(c // k,),
        in_specs=in_specs,
        out_specs=out_specs,
    )
    return pl.pallas_call(
        _body,
        grid_spec=grid_spec,
        out_shape=out_shapes,
        compiler_params=pltpu.CompilerParams(
            dimension_semantics=("arbitrary",)),
    )(order, cnt, camT, *ins)


def kernel(view_matrix, frustumplane, idx_tensor, feedback_visible_chunks_num,
           xyz, scale, rot, sh_0, sh_rest, opacity, cluster_origin,
           cluster_extend):
    c = cluster_origin.shape[0]
    n = xyz.shape[0]
    cs = n // c

    nrm = frustumplane[:, :3]
    dpl = frustumplane[:, 3]
    dist = (cluster_origin @ nrm.T + cluster_extend @ jnp.abs(nrm).T
            + dpl[None, :])
    mask = jnp.all(dist >= 0.0, axis=1)
    cnt = jnp.sum(mask.astype(jnp.int32))
    keys = jnp.where(mask, 0, 1) * c + jnp.arange(c)
    order = jnp.argsort(keys).astype(jnp.int32)
    visible_chunkid = jnp.take(idx_tensor, order)

    cam = view_matrix[3, :3]
    camT = jnp.broadcast_to(cam[:, None], (3, cs))

    xyz_v = xyz.reshape(c, cs, 3).transpose(0, 2, 1)
    sc_v = scale.reshape(c, cs, 3).transpose(0, 2, 1)
    rot_v = rot.reshape(c, cs, 4).transpose(0, 2, 1)
    op_v = opacity.reshape(c, cs, 1).transpose(0, 2, 1)
    sh0_c = sh_0.reshape(c, cs, 3).transpose(0, 2, 1)
    shr_v = sh_rest.transpose(2, 1, 0)

    packed = jnp.concatenate([xyz_v, sc_v, rot_v, op_v, sh0_c], axis=1)

    cx, csc, crt, col, cop = _gather_compute(
        order, cnt.reshape(1), camT, packed, shr_v)

    valid_length = cnt * cs
    return (visible_chunkid, cnt, valid_length,
            cx.transpose(0, 2, 1).reshape(n, 3),
            csc.transpose(0, 2, 1).reshape(n, 3),
            crt.transpose(0, 2, 1).reshape(n, 4),
            col.transpose(0, 2, 1).reshape(n, 3),
            cop.transpose(0, 2, 1).reshape(n, 1))


# speedup vs baseline: 2.0858x; 2.0858x over previous
import functools

import jax
import jax.numpy as jnp
from jax import lax
from jax.experimental import pallas as pl
from jax.experimental.pallas import tpu as pltpu
from jax.experimental.pallas import tpu_sc as plsc

_BA = 16
_K = 16


def _sc_compact(distT, idx_tensor):
    c = distT.shape[1]
    ng = c // 16
    mesh = plsc.VectorSubcoreMesh(core_axis_name="c", subcore_axis_name="s")

    @functools.partial(
        pl.kernel, mesh=mesh,
        out_type=[jax.ShapeDtypeStruct((c,), jnp.int32),
                  jax.ShapeDtypeStruct((c,), jnp.int32),
                  jax.ShapeDtypeStruct((16,), jnp.int32)],
        scratch_types=[pltpu.VMEM((6, c), jnp.float32),
                       pltpu.VMEM((c,), jnp.int32),
                       pltpu.VMEM((c,), jnp.int32),
                       pltpu.VMEM((c,), jnp.int32),
                       pltpu.VMEM((c,), jnp.int32),
                       pltpu.VMEM((16,), jnp.int32),
                       pltpu.VMEM((c,), jnp.int32),
                       pltpu.SemaphoreType.DMA,
                       pltpu.SemaphoreType.DMA],
    )
    def k(distT_hbm, idx_hbm, order_hbm, chunkid_hbm, cnt_hbm,
          dist_v, idx_v, vis_v, order_v, chunkid_v, cnt_v, pos_v, sem1, sem2):
        cid = lax.axis_index("c")
        sid = lax.axis_index("s")

        @pl.when(jnp.logical_and(cid == 0, sid == 0))
        def _():
            pltpu.sync_copy(distT_hbm, dist_v)
            pltpu.sync_copy(idx_hbm, idx_v)

            idxs = lax.iota(jnp.int32, 16)
            dn16 = lax.GatherDimensionNumbers(
                offset_dims=(), collapsed_slice_dims=(0,),
                start_index_map=(0,))

            def _perm(x, idx):
                return lax.gather(
                    x, idx[:, None], dn16, (1,),
                    mode=lax.GatherScatterMode.PROMISE_IN_BOUNDS)

            def _cumsum16(x):
                s = x
                for sh in (1, 2, 4, 8):
                    down = _perm(s, jnp.maximum(idxs - sh, 0))
                    s = s + jnp.where(idxs >= sh, down, 0)
                return s

            def _splat_last(x):
                return _perm(x, jnp.full((16,), 15, jnp.int32))

            cnt_v[...] = jnp.zeros((16,), jnp.int32)

            def pass_a(i, dummy):
                sl = pl.ds(i * 16, 16)
                ok = dist_v[0, sl] >= 0.0
                for p in range(1, 6):
                    ok = jnp.logical_and(ok, dist_v[p, sl] >= 0.0)
                oki = jnp.where(ok, 1, 0)
                vis_v[sl] = oki
                cnt_v[...] = cnt_v[...] + oki
                return dummy

            lax.fori_loop(0, ng, pass_a, 0)
            tot = _splat_last(_cumsum16(cnt_v[...]))
            cnt_v[...] = tot

            def pass_b(i, carry):
                sv, si = carry
                sl = pl.ds(i * 16, 16)
                visv = vis_v[sl]
                okm = visv > 0
                cv = _cumsum16(visv)
                ci = _cumsum16(1 - visv)
                pos = jnp.where(okm, sv + cv - 1, si + ci - 1)
                pos_v[sl] = pos
                order_v[sl] = idxs + (16 * i)
                nv = _splat_last(cv)
                return (sv + nv, si + (16 - nv))

            lax.fori_loop(0, ng, pass_b,
                          (jnp.zeros((16,), jnp.int32), tot))

            pltpu.async_copy(order_v, order_hbm.at[pos_v], sem1).wait()
            pltpu.async_copy(idx_v, chunkid_hbm.at[pos_v], sem2).wait()
            pltpu.sync_copy(cnt_v, cnt_hbm)

    return k(distT, idx_tensor)


def _basis_rows(x, y, z):
    xx = x * x; yy = y * y; zz = z * z
    xy = x * y; yz = y * z; xz = x * z
    return [
        -0.4886025119029199 * y,
        0.4886025119029199 * z,
        -0.4886025119029199 * x,
        1.0925484305920792 * xy,
        -1.0925484305920792 * yz,
        0.31539156525252005 * (2.0 * zz - xx - yy),
        -1.0925484305920792 * xz,
        0.5462742152960396 * (xx - yy),
        -0.5900435899266435 * y * (3.0 * xx - yy),
        2.890611442640554 * xy * z,
        -0.4570457994644658 * y * (4.0 * zz - xx - yy),
        0.3731763325901154 * z * (2.0 * zz - 3.0 * xx - 3.0 * yy),
        -0.4570457994644658 * x * (4.0 * zz - xx - yy),
        1.445305721320277 * z * (xx - yy),
        -0.5900435899266435 * x * (xx - yy - zz),
    ]


def _math_body(camT_ref, xyz_ref, sc_ref, rot_ref, sh0_ref, shr_ref, op_ref,
               out_ref):
    ba, _, cs = xyz_ref.shape
    xyz = xyz_ref[...]
    rot = rot_ref[...]

    out_ref[:, 0:3] = xyz
    out_ref[:, 3:6] = jnp.exp(sc_ref[...])
    qn = jnp.sqrt(jnp.sum(rot * rot, axis=1, keepdims=True))
    out_ref[:, 6:10] = rot * (1.0 / (qn + 1e-8))
    out_ref[:, 13:14] = 1.0 / (1.0 + jnp.exp(-op_ref[...]))

    d = xyz - camT_ref[...][None]
    dn = jnp.sqrt(jnp.sum(d * d, axis=1, keepdims=True))
    dirs = d * (1.0 / (dn + 1e-8))

    shr = shr_ref[...]
    sh0 = sh0_ref[...]
    for jj in range(ba):
        lo, hi = jj * cs, (jj + 1) * cs
        dj = dirs[jj]
        x = dj[0:1]; y = dj[1:2]; z = dj[2:3]
        basis = jnp.concatenate(_basis_rows(x, y, z), axis=0)
        rows = [
            jnp.sum(basis * shr[ch, :, lo:hi], axis=0, keepdims=True)
            + 0.28209479177387814 * sh0[ch, :, lo:hi]
            for ch in range(3)
        ]
        colT = jnp.maximum(jnp.concatenate(rows, axis=0) + 0.5, 0.0)
        out_ref[jj, 10:13] = colT


def _dense_math(camT, xyz_v, sc_v, rot_v, sh0_v, shr_v, op_v):
    c, _, cs = xyz_v.shape
    ba = _BA if c % _BA == 0 else 1
    idx_c = lambda i: (i, 0, 0)
    idx_n = lambda i: (0, 0, i)
    return pl.pallas_call(
        _math_body,
        grid=(c // ba,),
        in_specs=[
            pl.BlockSpec((3, cs), lambda i: (0, 0)),
            pl.BlockSpec((ba, 3, cs), idx_c),
            pl.BlockSpec((ba, 3, cs), idx_c),
            pl.BlockSpec((ba, 4, cs), idx_c),
            pl.BlockSpec((3, 1, ba * cs), idx_n),
            pl.BlockSpec((3, 15, ba * cs), idx_n),
            pl.BlockSpec((ba, 1, cs), idx_c),
        ],
        out_specs=pl.BlockSpec((ba, 14, cs), idx_c),
        out_shape=jax.ShapeDtypeStruct((c, 14, cs), jnp.float32),
        compiler_params=pltpu.CompilerParams(
            dimension_semantics=("arbitrary",)),
    )(camT, xyz_v, sc_v, rot_v, sh0_v, shr_v, op_v)


def _compact_body(order_ref, cnt_ref, *refs):
    k = len(refs) - 5
    ins = refs[:k]
    cx_ref, csc_ref, crt_ref, col_ref, cop_ref = refs[k:]
    i = pl.program_id(0)
    for j in range(k):
        pk_ref = ins[j]
        valid = (k * i + j) < cnt_ref[0]
        pk = pk_ref[0]
        cx_ref[j] = jnp.where(valid, pk[0:3], 0.0)
        csc_ref[j] = jnp.where(valid, pk[3:6], 1.0)
        crt_ref[j] = jnp.where(valid, pk[6:10], 0.0)
        col_ref[j] = jnp.where(valid, pk[10:13], 0.5)
        cop_ref[j] = jnp.where(valid, pk[13:14], 0.5)


def _compact(order, cnt, packed):
    c, _, cs = packed.shape
    k = _K if c % _K == 0 else 1

    chunk_specs = []
    ins = []
    for j in range(k):
        idx3 = (lambda jj: lambda i, o, n: (o[k * i + jj], 0, 0))(j)
        chunk_specs += [pl.BlockSpec((1, 14, cs), idx3)]
        ins += [packed]

    out_specs = [pl.BlockSpec((k, d, cs), lambda i, o, n: (i, 0, 0))
                 for d in (3, 3, 4, 3, 1)]
    out_shapes = [jax.ShapeDtypeStruct((c, d, cs), jnp.float32)
                  for d in (3, 3, 4, 3, 1)]

    grid_spec = pltpu.PrefetchScalarGridSpec(
        num_scalar_prefetch=2,
        grid=(c // k,),
        in_specs=chunk_specs,
        out_specs=out_specs,
    )
    return pl.pallas_call(
        _compact_body,
        grid_spec=grid_spec,
        out_shape=out_shapes,
        compiler_params=pltpu.CompilerParams(
            dimension_semantics=("arbitrary",)),
    )(order, cnt, *ins)


def kernel(view_matrix, frustumplane, idx_tensor, feedback_visible_chunks_num,
           xyz, scale, rot, sh_0, sh_rest, opacity, cluster_origin,
           cluster_extend):
    c = cluster_origin.shape[0]
    n = xyz.shape[0]
    cs = n // c

    nrm = frustumplane[:, :3]
    dpl = frustumplane[:, 3]
    dist = (cluster_origin @ nrm.T + cluster_extend @ jnp.abs(nrm).T
            + dpl[None, :])
    order, visible_chunkid, cnt_vec = _sc_compact(
        dist.T, idx_tensor.astype(jnp.int32))
    cnt = cnt_vec[0]

    cam = view_matrix[3, :3]
    camT = jnp.broadcast_to(cam[:, None], (3, cs))

    xyz_v = xyz.reshape(c, cs, 3).transpose(0, 2, 1)
    sc_v = scale.reshape(c, cs, 3).transpose(0, 2, 1)
    rot_v = rot.reshape(c, cs, 4).transpose(0, 2, 1)
    op_v = opacity.reshape(c, cs, 1).transpose(0, 2, 1)
    sh0_v = sh_0.transpose(2, 1, 0)
    shr_v = sh_rest.transpose(2, 1, 0)

    packed = _dense_math(camT, xyz_v, sc_v, rot_v, sh0_v, shr_v, op_v)
    cx, csc, crt, col, cop = _compact(order, cnt_vec[0:1], packed)

    valid_length = cnt * cs
    return (visible_chunkid, cnt, valid_length,
            cx.transpose(0, 2, 1).reshape(n, 3),
            csc.transpose(0, 2, 1).reshape(n, 3),
            crt.transpose(0, 2, 1).reshape(n, 4),
            col.transpose(0, 2, 1).reshape(n, 3),
            cop.transpose(0, 2, 1).reshape(n, 1))
